# Initial kernel scaffold; baseline (speedup 1.0000x reference)
import jax
import jax.numpy as jnp
from jax.experimental import pallas as pl


def kernel(vertices, normals, params):
    raise NotImplementedError("write your pallas kernel here")


# speedup vs baseline: 1.2339x; 1.2339x over previous
import jax
import jax.numpy as jnp
from jax.experimental import pallas as pl
from jax.experimental.pallas import tpu as pltpu


def _pad_rows(a, m):
    n = a.shape[0]
    npad = -n % m
    if npad == 0:
        return a
    return jnp.concatenate([a, jnp.full((npad,) + a.shape[1:], 1e3, a.dtype)], axis=0)


def _dist_body(src_ref, dstt_ref, sqs_ref, sqd_ref, out_ref):
    out_ref[...] = (sqs_ref[...] + sqd_ref[...]
                    - 2.0 * jnp.dot(src_ref[...], dstt_ref[...]))


def _sqdist_pallas(src, dst):
    N, M = src.shape[0], dst.shape[0]
    srcp = _pad_rows(src, 256)
    dstp = _pad_rows(dst, 256)
    Np, Mp = srcp.shape[0], dstp.shape[0]
    bm = min(Mp, 2048)
    sqs = jnp.sum(srcp ** 2, -1)[:, None]
    sqd = jnp.sum(dstp ** 2, -1)[None, :]
    out = pl.pallas_call(
        _dist_body,
        grid=(Np // 256, Mp // bm),
        in_specs=[
            pl.BlockSpec((256, 3), lambda i, j: (i, 0)),
            pl.BlockSpec((3, bm), lambda i, j: (0, j)),
            pl.BlockSpec((256, 1), lambda i, j: (i, 0)),
            pl.BlockSpec((1, bm), lambda i, j: (0, j)),
        ],
        out_specs=pl.BlockSpec((256, bm), lambda i, j: (i, j)),
        out_shape=jax.ShapeDtypeStruct((Np, Mp), jnp.float32),
        interpret=_INTERPRET,
    )(srcp, dstp.T, sqs, sqd)
    return out


def _sqdist(src, dst):
    N, M = src.shape[0], dst.shape[0]
    return _sqdist_pallas(src, dst)[:N, :M]


_INTERPRET = False


def _fps_body(xyzp_ref, xyzt_ref, sqn_ref, sqnc_ref, nvalid_ref, idx_ref, dist_ref):
    Np = sqn_ref.shape[1]
    npoint = idx_ref.shape[0]
    gidx = jax.lax.broadcasted_iota(jnp.int32, (1, Np), 1)
    nv = nvalid_ref[0, 0]
    dist_ref[...] = jnp.where(gidx < nv, jnp.float32(1e10), jnp.float32(-1.0))

    def body(i, farthest):
        idx_ref[pl.ds(i, 1), :] = jnp.full((1, 1), farthest, jnp.int32)
        xf = xyzp_ref[pl.ds(farthest, 1), :]
        sqn_f = sqnc_ref[pl.ds(farthest, 1), :][0, 0]
        lhs = jnp.broadcast_to(xf, (8, 3))
        mm = jax.lax.dot_general(lhs, xyzt_ref[...],
                                 (((1,), (0,)), ((), ())),
                                 precision=jax.lax.Precision.DEFAULT)[0:1, :]
        row = (sqn_f + sqn_ref[...]) - 2.0 * mm
        nd = jnp.minimum(dist_ref[...], row)
        dist_ref[...] = nd
        m = jnp.max(nd)
        fa = jnp.min(jnp.where(nd == m, gidx, jnp.int32(2 ** 30)))
        return fa

    jax.lax.fori_loop(0, npoint, body, jnp.int32(0))


def _fps_pallas(xyz, npoint):
    N = xyz.shape[0]
    Np = N + (-N % 256)
    xyzp = _pad_rows(xyz, 256)
    sqn = jnp.sum(xyzp ** 2, -1)
    idx = pl.pallas_call(
        _fps_body,
        in_specs=[
            pl.BlockSpec((Np, 3), lambda: (0, 0)),
            pl.BlockSpec((3, Np), lambda: (0, 0)),
            pl.BlockSpec((1, Np), lambda: (0, 0)),
            pl.BlockSpec((Np, 1), lambda: (0, 0)),
            pl.BlockSpec((1, 1), lambda: (0, 0), memory_space=pltpu.SMEM),
        ],
        out_specs=pl.BlockSpec((npoint, 1), lambda: (0, 0)),
        out_shape=jax.ShapeDtypeStruct((npoint, 1), jnp.int32),
        scratch_shapes=[pltpu.VMEM((1, Np), jnp.float32)],
        interpret=_INTERPRET,
    )(xyzp, xyzp.T, sqn[None, :], sqn[:, None],
      jnp.full((1, 1), N, jnp.int32))
    return idx[:, 0]


def _fps(dists, npoint):
    N = dists.shape[0]

    def body(i, state):
        centroids, distance, farthest = state
        centroids = centroids.at[i].set(farthest)
        d = dists[farthest]
        distance = jnp.minimum(distance, d)
        farthest = jnp.argmax(distance).astype(jnp.int32)
        return centroids, distance, farthest

    init = (jnp.zeros((npoint,), jnp.int32), jnp.full((N,), 1e10, jnp.float32), jnp.int32(0))
    centroids, _, _ = jax.lax.fori_loop(0, npoint, body, init)
    return centroids


def _bn(x, gamma, beta, axes):
    mean = jnp.mean(x, axis=axes, keepdims=True)
    var = jnp.var(x, axis=axes, keepdims=True)
    return gamma * (x - mean) / jnp.sqrt(var + 1e-5) + beta


def _sa(xyz, points, dists, radius, nsample, layers):
    N = xyz.shape[0]
    npoint = N // 4
    fps_idx = _fps_pallas(xyz, npoint)
    new_xyz = xyz[fps_idx]
    sqrdists = dists[fps_idx]
    group_idx = jnp.broadcast_to(jnp.arange(N), (npoint, N))
    group_idx = jnp.where(sqrdists > radius ** 2, N, group_idx)
    group_idx = jnp.sort(group_idx, axis=-1)[:, :nsample]
    first = group_idx[:, :1]
    group_idx = jnp.where(group_idx == N, first, group_idx)
    grouped_xyz = xyz[group_idx] - new_xyz[:, None, :]
    feat = jnp.concatenate([points[group_idx], grouped_xyz], axis=-1)
    for lyr in layers:
        feat = feat @ lyr["W"] + lyr["b"]
        feat = _bn(feat, lyr["gamma"], lyr["beta"], axes=(0, 1))
        feat = jax.nn.relu(feat)
    new_points = jnp.max(feat, axis=1)
    new_dists = dists[fps_idx][:, fps_idx]
    return new_xyz, new_points, fps_idx, new_dists


def _fp(points1, points2, fps_idx, dists, layers):
    d = dists[:, fps_idx]
    neg_vals, idx = jax.lax.top_k(-d, 3)
    dist3 = -neg_vals
    recip = 1.0 / (dist3 + 1e-8)
    weight = recip / jnp.sum(recip, axis=-1, keepdims=True)
    interpolated = jnp.sum(points2[idx] * weight[..., None], axis=1)
    feat = interpolated if points1 is None else jnp.concatenate([points1, interpolated], axis=-1)
    for lyr in layers:
        feat = feat @ lyr["W"] + lyr["b"]
        feat = _bn(feat, lyr["gamma"], lyr["beta"], axes=(0,))
        feat = jax.nn.relu(feat)
    return feat


def kernel(vertices, normals, params):
    dists = _sqdist(vertices, vertices)
    l0_points = jnp.concatenate([vertices, normals], axis=-1)
    l1_xyz, l1_points, fps1, d2 = _sa(vertices, l0_points, dists, 0.06, 32, params["sa1"])
    l2_xyz, l2_points, fps2, d3 = _sa(l1_xyz, l1_points, d2, 0.1, 32, params["sa2"])
    l3_xyz, l3_points, fps3, d4 = _sa(l2_xyz, l2_points, d3, 0.14, 32, params["sa3"])
    l4_xyz, l4_points, fps4, d5 = _sa(l3_xyz, l3_points, d4, 0.18, 32, params["sa4"])
    l3_points = _fp(l3_points, l4_points, fps4, d4, params["fp4"])
    l2_points = _fp(l2_points, l3_points, fps3, d3, params["fp3"])
    l1_points = _fp(l1_points, l2_points, fps2, d2, params["fp2"])
    l0_points = _fp(None, l1_points, fps1, dists, params["fp1"])
    x = l0_points @ params["conv1"]["W"] + params["conv1"]["b"]
    mean = jnp.mean(x, axis=0, keepdims=True)
    var = jnp.var(x, axis=0, keepdims=True)
    x = (x - mean) / jnp.sqrt(var + 1e-5)
    x = jax.nn.relu(x)
    return x[None]


# speedup vs baseline: 1.2341x; 1.0002x over previous
import jax
import jax.numpy as jnp
from jax.experimental import pallas as pl
from jax.experimental.pallas import tpu as pltpu

_BIG = 2 ** 30


def _pad_rows(a, m, value=1e3):
    n = a.shape[0]
    npad = -n % m
    if npad == 0:
        return a
    return jnp.concatenate(
        [a, jnp.full((npad,) + a.shape[1:], value, a.dtype)], axis=0)


def _dist_body(src_ref, dstt_ref, sqs_ref, sqd_ref, out_ref):
    out_ref[...] = (sqs_ref[...] + sqd_ref[...]
                    - 2.0 * jnp.dot(src_ref[...], dstt_ref[...]))


def _sqdist_pallas(src, dst):
    N, M = src.shape[0], dst.shape[0]
    srcp = _pad_rows(src, 256)
    dstp = _pad_rows(dst, 256)
    Np, Mp = srcp.shape[0], dstp.shape[0]
    bm = min(Mp, 2048)
    sqs = jnp.sum(srcp ** 2, -1)[:, None]
    sqd = jnp.sum(dstp ** 2, -1)[None, :]
    out = pl.pallas_call(
        _dist_body,
        grid=(Np // 256, Mp // bm),
        in_specs=[
            pl.BlockSpec((256, 3), lambda i, j: (i, 0)),
            pl.BlockSpec((3, bm), lambda i, j: (0, j)),
            pl.BlockSpec((256, 1), lambda i, j: (i, 0)),
            pl.BlockSpec((1, bm), lambda i, j: (0, j)),
        ],
        out_specs=pl.BlockSpec((256, bm), lambda i, j: (i, j)),
        out_shape=jax.ShapeDtypeStruct((Np, Mp), jnp.float32),
    )(srcp, dstp.T, sqs, sqd)
    return out


def _sqdist(src, dst):
    N, M = src.shape[0], dst.shape[0]
    return _sqdist_pallas(src, dst)[:N, :M]


def _fps_body(xyzp_ref, xyzt_ref, sqn_ref, sqnc_ref, nvalid_ref, idx_ref, dist_ref):
    Np = sqn_ref.shape[1]
    npoint = idx_ref.shape[0]
    gidx = jax.lax.broadcasted_iota(jnp.int32, (1, Np), 1)
    nv = nvalid_ref[0, 0]
    dist_ref[...] = jnp.where(gidx < nv, jnp.float32(1e10), jnp.float32(-1.0))

    def body(i, farthest):
        idx_ref[pl.ds(i, 1), :] = jnp.full((1, 1), farthest, jnp.int32)
        xf = xyzp_ref[pl.ds(farthest, 1), :]
        sqn_f = sqnc_ref[pl.ds(farthest, 1), :][0, 0]
        lhs = jnp.broadcast_to(xf, (8, 3))
        mm = jax.lax.dot_general(lhs, xyzt_ref[...],
                                 (((1,), (0,)), ((), ())),
                                 precision=jax.lax.Precision.DEFAULT)[0:1, :]
        row = (sqn_f + sqn_ref[...]) - 2.0 * mm
        nd = jnp.minimum(dist_ref[...], row)
        dist_ref[...] = nd
        m = jnp.max(nd)
        fa = jnp.min(jnp.where(nd == m, gidx, _BIG))
        return fa

    jax.lax.fori_loop(0, npoint, body, jnp.int32(0))


def _fps_pallas(xyz, npoint):
    N = xyz.shape[0]
    xyzp = _pad_rows(xyz, 256)
    Np = xyzp.shape[0]
    sqn = jnp.sum(xyzp ** 2, -1)
    idx = pl.pallas_call(
        _fps_body,
        in_specs=[
            pl.BlockSpec((Np, 3), lambda: (0, 0)),
            pl.BlockSpec((3, Np), lambda: (0, 0)),
            pl.BlockSpec((1, Np), lambda: (0, 0)),
            pl.BlockSpec((Np, 1), lambda: (0, 0)),
            pl.BlockSpec((1, 1), lambda: (0, 0), memory_space=pltpu.SMEM),
        ],
        out_specs=pl.BlockSpec((npoint, 1), lambda: (0, 0)),
        out_shape=jax.ShapeDtypeStruct((npoint, 1), jnp.int32),
        scratch_shapes=[pltpu.VMEM((1, Np), jnp.float32)],
    )(xyzp, xyzp.T, sqn[None, :], sqn[:, None],
      jnp.full((1, 1), N, jnp.int32))
    return idx[:, 0]


def _batch_norm(x, gamma, beta, axes):
    mean = jnp.mean(x, axis=axes, keepdims=True)
    var = jnp.var(x, axis=axes, keepdims=True)
    return gamma * (x - mean) / jnp.sqrt(var + 1e-5) + beta


def _set_abstraction(xyz, points, dists, radius, nsample, layers):
    N = xyz.shape[0]
    npoint = N // 4
    fps_idx = _fps_pallas(xyz, npoint)
    new_xyz = xyz[fps_idx]
    sqrdists = dists[fps_idx]
    group_idx = jnp.broadcast_to(jnp.arange(N), (npoint, N))
    group_idx = jnp.where(sqrdists > radius ** 2, N, group_idx)
    group_idx = jnp.sort(group_idx, axis=-1)[:, :nsample]
    first = group_idx[:, :1]
    group_idx = jnp.where(group_idx == N, first, group_idx)
    grouped_xyz = xyz[group_idx] - new_xyz[:, None, :]
    feat = jnp.concatenate([points[group_idx], grouped_xyz], axis=-1)
    for lyr in layers:
        feat = feat @ lyr["W"] + lyr["b"]
        feat = _batch_norm(feat, lyr["gamma"], lyr["beta"], axes=(0, 1))
        feat = jax.nn.relu(feat)
    new_points = jnp.max(feat, axis=1)
    new_dists = dists[fps_idx][:, fps_idx]
    return new_xyz, new_points, fps_idx, new_dists


def _feature_propagation(points1, points2, fps_idx, dists, layers):
    d = dists[:, fps_idx]
    neg_vals, idx = jax.lax.top_k(-d, 3)
    dist3 = -neg_vals
    recip = 1.0 / (dist3 + 1e-8)
    weight = recip / jnp.sum(recip, axis=-1, keepdims=True)
    interpolated = jnp.sum(points2[idx] * weight[..., None], axis=1)
    feat = interpolated if points1 is None else jnp.concatenate([points1, interpolated], axis=-1)
    for lyr in layers:
        feat = feat @ lyr["W"] + lyr["b"]
        feat = _batch_norm(feat, lyr["gamma"], lyr["beta"], axes=(0,))
        feat = jax.nn.relu(feat)
    return feat


def kernel(vertices, normals, params):
    dists = _sqdist(vertices, vertices)
    l0_points = jnp.concatenate([vertices, normals], axis=-1)
    l1_xyz, l1_points, fps1, d2 = _set_abstraction(vertices, l0_points, dists, 0.06, 32, params["sa1"])
    l2_xyz, l2_points, fps2, d3 = _set_abstraction(l1_xyz, l1_points, d2, 0.1, 32, params["sa2"])
    l3_xyz, l3_points, fps3, d4 = _set_abstraction(l2_xyz, l2_points, d3, 0.14, 32, params["sa3"])
    l4_xyz, l4_points, fps4, d5 = _set_abstraction(l3_xyz, l3_points, d4, 0.18, 32, params["sa4"])
    l3_points = _feature_propagation(l3_points, l4_points, fps4, d4, params["fp4"])
    l2_points = _feature_propagation(l2_points, l3_points, fps3, d3, params["fp3"])
    l1_points = _feature_propagation(l1_points, l2_points, fps2, d2, params["fp2"])
    l0_points = _feature_propagation(None, l1_points, fps1, dists, params["fp1"])
    x = l0_points @ params["conv1"]["W"] + params["conv1"]["b"]
    mean = jnp.mean(x, axis=0, keepdims=True)
    var = jnp.var(x, axis=0, keepdims=True)
    x = (x - mean) / jnp.sqrt(var + 1e-5)
    x = jax.nn.relu(x)
    return x[None]
